# Initial kernel scaffold; baseline (speedup 1.0000x reference)
import jax
import jax.numpy as jnp
from jax.experimental import pallas as pl


def kernel(target, observed_mask, sample_id, variate_id):
    raise NotImplementedError("write your pallas kernel here")


# speedup vs baseline: 242.7412x; 242.7412x over previous
import functools

import jax
import jax.numpy as jnp
from jax import lax
from jax.experimental import pallas as pl
from jax.experimental.pallas import tpu as pltpu
from jax.experimental.pallas import tpu_sc as plsc

MINIMUM_SCALE = 1e-05

_B, _L, _D = 16, 4096, 64
_NG = 64
_C = 1024
_NCHUNK = _L // _C
_HALF = _D // 2


def _sc_body(tgt_hbm, msk_hbm, gk_hbm, nv_hbm, out_hbm,
             tgt_v, msk_v, out_v, gk_v, acc_s, acc_c, nv_v):
    c = lax.axis_index("c")
    s = lax.axis_index("s")
    col = _HALF * c

    pltpu.sync_copy(gk_hbm.at[s], gk_v)
    pltpu.sync_copy(nv_hbm, nv_v)

    zeros16 = jnp.zeros((16,), jnp.float32)

    def zero_body(g, carry):
        acc_s[g, pl.ds(0, 16)] = zeros16
        acc_s[g, pl.ds(16, 16)] = zeros16
        acc_c[g, pl.ds(0, 16)] = zeros16
        acc_c[g, pl.ds(16, 16)] = zeros16
        return carry

    lax.fori_loop(0, _NG, zero_body, 0)

    for ch in range(_NCHUNK):
        t0 = ch * _C
        pltpu.sync_copy(tgt_hbm.at[s, pl.ds(t0, _C), pl.ds(col, _HALF)], tgt_v)
        pltpu.sync_copy(msk_hbm.at[s, pl.ds(t0, _C), pl.ds(col, _HALF)], msk_v)

        def p1_body(tg, carry):
            base = tg * 16
            gkv = gk_v[pl.ds(t0 + base, 16)]
            for j in range(16):
                g = gkv[j]
                t = base + j
                m0 = msk_v[t, pl.ds(0, 16)]
                m1 = msk_v[t, pl.ds(16, 16)]
                a0 = jnp.abs(tgt_v[t, pl.ds(0, 16)]) * m0
                a1 = jnp.abs(tgt_v[t, pl.ds(16, 16)]) * m1
                acc_s[g, pl.ds(0, 16)] = acc_s[g, pl.ds(0, 16)] + a0
                acc_s[g, pl.ds(16, 16)] = acc_s[g, pl.ds(16, 16)] + a1
                acc_c[g, pl.ds(0, 16)] = acc_c[g, pl.ds(0, 16)] + m0
                acc_c[g, pl.ds(16, 16)] = acc_c[g, pl.ds(16, 16)] + m1
            return carry

        lax.fori_loop(0, _C // 16, p1_body, 0)

    nvec = nv_v[...]
    ones16 = jnp.full((16,), 1.0, jnp.float32)
    eps16 = jnp.full((16,), MINIMUM_SCALE, jnp.float32)

    def fin_body(g, carry):
        gvec = jnp.full((16,), 1, jnp.int32) * g
        pad = gvec < nvec
        for j in (0, 16):
            sm = acc_s[g, pl.ds(j, 16)]
            cnt = acc_c[g, pl.ds(j, 16)]
            iszero = cnt == 0.0
            sc = sm / jnp.where(iszero, ones16, cnt)
            sc = jnp.where(iszero, zeros16, sc)
            sc = jnp.maximum(sc, eps16)
            sc = jnp.where(pad, ones16, sc)
            acc_s[g, pl.ds(j, 16)] = sc
        return carry

    lax.fori_loop(0, _NG, fin_body, 0)

    for ch in range(_NCHUNK):
        t0 = ch * _C

        def p2_body(tg, carry):
            base = tg * 16
            gkv = gk_v[pl.ds(t0 + base, 16)]
            for j in range(16):
                g = gkv[j]
                t = base + j
                out_v[t, pl.ds(0, 16)] = acc_s[g, pl.ds(0, 16)]
                out_v[t, pl.ds(16, 16)] = acc_s[g, pl.ds(16, 16)]
            return carry

        lax.fori_loop(0, _C // 16, p2_body, 0)
        pltpu.sync_copy(out_v, out_hbm.at[s, pl.ds(t0, _C), pl.ds(col, _HALF)])


@functools.partial(jax.jit, static_argnums=())
def _sc_scale(target, maskf, gk, nv_arr):
    mesh = plsc.VectorSubcoreMesh(core_axis_name="c", subcore_axis_name="s")
    fn = functools.partial(
        pl.kernel,
        mesh=mesh,
        compiler_params=pltpu.CompilerParams(use_tc_tiling_on_sc=False),
        out_type=jax.ShapeDtypeStruct((_B, _L, _D), jnp.float32),
        scratch_types=[
            pltpu.VMEM((_C, _HALF), jnp.float32),
            pltpu.VMEM((_C, _HALF), jnp.float32),
            pltpu.VMEM((_C, _HALF), jnp.float32),
            pltpu.VMEM((_L,), jnp.int32),
            pltpu.VMEM((_NG, _HALF), jnp.float32),
            pltpu.VMEM((_NG, _HALF), jnp.float32),
            pltpu.VMEM((16,), jnp.int32),
        ],
    )(_sc_body)
    return fn(target, maskf, gk, nv_arr)


def kernel(target, observed_mask, sample_id, variate_id):
    nv = (variate_id.max() + 1).astype(jnp.int32)
    gk = sample_id.astype(jnp.int32) * nv + variate_id.astype(jnp.int32)
    maskf = observed_mask.astype(jnp.float32)
    nv_arr = jnp.full((16,), nv, jnp.int32)
    scale = _sc_scale(target, maskf, gk, nv_arr)
    loc = jnp.zeros_like(target)
    return (loc, scale)


# speedup vs baseline: 273.5480x; 1.1269x over previous
import functools

import jax
import jax.numpy as jnp
from jax import lax
from jax.experimental import pallas as pl
from jax.experimental.pallas import tpu as pltpu
from jax.experimental.pallas import tpu_sc as plsc

MINIMUM_SCALE = 1e-05

_B, _L, _D = 16, 4096, 64
_NG = 64
_C = 512
_NCHUNK = _L // _C
_HALF = _D // 2


def _sc_body(tgt_hbm, msk_hbm, gk_hbm, nv_hbm, out_hbm,
             tgt_v, msk_v, out_v, gk_v, acc_s, acc_c, nv_v,
             tsem, msem, osem):
    c = lax.axis_index("c")
    s = lax.axis_index("s")
    col = _HALF * c

    def in_slab(ch):
        return (s, pl.ds(ch * _C, _C), pl.ds(col, _HALF))

    t_cp = [None] * _NCHUNK
    m_cp = [None] * _NCHUNK
    t_cp[0] = pltpu.async_copy(tgt_hbm.at[in_slab(0)], tgt_v.at[0], tsem[0])
    m_cp[0] = pltpu.async_copy(msk_hbm.at[in_slab(0)], msk_v.at[0], msem[0])

    pltpu.sync_copy(gk_hbm.at[s], gk_v)
    pltpu.sync_copy(nv_hbm, nv_v)

    zeros16 = jnp.zeros((16,), jnp.float32)

    def zero_body(g, carry):
        acc_s[g, pl.ds(0, 16)] = zeros16
        acc_s[g, pl.ds(16, 16)] = zeros16
        acc_c[g, pl.ds(0, 16)] = zeros16
        acc_c[g, pl.ds(16, 16)] = zeros16
        return carry

    lax.fori_loop(0, _NG, zero_body, 0)

    for ch in range(_NCHUNK):
        buf = ch % 2
        if ch + 1 < _NCHUNK:
            nbuf = (ch + 1) % 2
            t_cp[ch + 1] = pltpu.async_copy(
                tgt_hbm.at[in_slab(ch + 1)], tgt_v.at[nbuf], tsem[nbuf])
            m_cp[ch + 1] = pltpu.async_copy(
                msk_hbm.at[in_slab(ch + 1)], msk_v.at[nbuf], msem[nbuf])
        t_cp[ch].wait()
        m_cp[ch].wait()
        t0 = ch * _C

        def p1_body(tg, carry):
            base = tg * 16
            gkv = gk_v[pl.ds(t0 + base, 16)]
            for j in range(16):
                g = gkv[j]
                t = base + j
                m0 = msk_v[buf, t, pl.ds(0, 16)]
                m1 = msk_v[buf, t, pl.ds(16, 16)]
                a0 = jnp.abs(tgt_v[buf, t, pl.ds(0, 16)]) * m0
                a1 = jnp.abs(tgt_v[buf, t, pl.ds(16, 16)]) * m1
                plsc.addupdate(acc_s.at[g, pl.ds(0, 16)], a0)
                plsc.addupdate(acc_s.at[g, pl.ds(16, 16)], a1)
                plsc.addupdate(acc_c.at[g, pl.ds(0, 16)], m0)
                plsc.addupdate(acc_c.at[g, pl.ds(16, 16)], m1)
            return carry

        lax.fori_loop(0, _C // 16, p1_body, 0)

    nvec = nv_v[...]
    ones16 = jnp.full((16,), 1.0, jnp.float32)
    eps16 = jnp.full((16,), MINIMUM_SCALE, jnp.float32)

    def fin_body(g, carry):
        gvec = jnp.full((16,), 1, jnp.int32) * g
        pad = gvec < nvec
        for j in (0, 16):
            sm = acc_s[g, pl.ds(j, 16)]
            cnt = acc_c[g, pl.ds(j, 16)]
            iszero = cnt == 0.0
            sc = sm / jnp.where(iszero, ones16, cnt)
            sc = jnp.where(iszero, zeros16, sc)
            sc = jnp.maximum(sc, eps16)
            sc = jnp.where(pad, ones16, sc)
            acc_s[g, pl.ds(j, 16)] = sc
        return carry

    lax.fori_loop(0, _NG, fin_body, 0)

    o_cp = [None] * _NCHUNK
    for ch in range(_NCHUNK):
        buf = ch % 2
        if ch >= 2:
            o_cp[ch - 2].wait()
        t0 = ch * _C

        def p2_body(tg, carry):
            base = tg * 16
            gkv = gk_v[pl.ds(t0 + base, 16)]
            for j in range(16):
                g = gkv[j]
                t = base + j
                out_v[buf, t, pl.ds(0, 16)] = acc_s[g, pl.ds(0, 16)]
                out_v[buf, t, pl.ds(16, 16)] = acc_s[g, pl.ds(16, 16)]
            return carry

        lax.fori_loop(0, _C // 16, p2_body, 0)
        o_cp[ch] = pltpu.async_copy(
            out_v.at[buf], out_hbm.at[in_slab(ch)], osem[buf])
    o_cp[_NCHUNK - 2].wait()
    o_cp[_NCHUNK - 1].wait()


@functools.partial(jax.jit, static_argnums=())
def _sc_scale(target, maskf, gk, nv_arr):
    mesh = plsc.VectorSubcoreMesh(core_axis_name="c", subcore_axis_name="s")
    fn = functools.partial(
        pl.kernel,
        mesh=mesh,
        compiler_params=pltpu.CompilerParams(use_tc_tiling_on_sc=False),
        out_type=jax.ShapeDtypeStruct((_B, _L, _D), jnp.float32),
        scratch_types=[
            pltpu.VMEM((2, _C, _HALF), jnp.float32),
            pltpu.VMEM((2, _C, _HALF), jnp.float32),
            pltpu.VMEM((2, _C, _HALF), jnp.float32),
            pltpu.VMEM((_L,), jnp.int32),
            pltpu.VMEM((_NG, _HALF), jnp.float32),
            pltpu.VMEM((_NG, _HALF), jnp.float32),
            pltpu.VMEM((16,), jnp.int32),
            [pltpu.SemaphoreType.DMA] * 2,
            [pltpu.SemaphoreType.DMA] * 2,
            [pltpu.SemaphoreType.DMA] * 2,
        ],
    )(_sc_body)
    return fn(target, maskf, gk, nv_arr)


def kernel(target, observed_mask, sample_id, variate_id):
    nv = (variate_id.max() + 1).astype(jnp.int32)
    gk = sample_id.astype(jnp.int32) * nv + variate_id.astype(jnp.int32)
    maskf = observed_mask.astype(jnp.float32)
    nv_arr = jnp.full((16,), nv, jnp.int32)
    scale = _sc_scale(target, maskf, gk, nv_arr)
    loc = jnp.zeros_like(target)
    return (loc, scale)
